# Initial kernel scaffold; baseline (speedup 1.0000x reference)
import jax
import jax.numpy as jnp
from jax.experimental import pallas as pl


def kernel(x, table, W1, b1, W2, b2):
    raise NotImplementedError("write your pallas kernel here")


# speedup vs baseline: 6.5096x; 6.5096x over previous
import functools

import jax
import jax.numpy as jnp
from jax import lax
from jax.experimental import pallas as pl
from jax.experimental.pallas import tpu as pltpu
from jax.experimental.pallas import tpu_sc as plsc

VOCAB = 100000
D = 128
HIDDEN = 64
NCLASS = 10
B = 4096
SEQ = 50

NC = 2
NS = 16
NW = NC * NS
RPW = B // NW
RCHUNK = 2
K = RCHUNK * SEQ
NCHUNK = RPW // RCHUNK
ROWS_PER_CORE = B // NC

_mesh = plsc.VectorSubcoreMesh(
    core_axis_name="c", subcore_axis_name="s", num_cores=NC, num_subcores=NS
)


@functools.partial(
    pl.kernel,
    out_type=jax.ShapeDtypeStruct((B, D), jnp.float32),
    mesh=_mesh,
    scratch_types=dict(
        idx_v=pltpu.VMEM((NCHUNK, K), jnp.int32),
        sidx_v=pltpu.VMEM((NCHUNK, K), jnp.int32),
        buf=pltpu.VMEM((K, D), jnp.float32),
        zbuf=pltpu.VMEM((RPW, D), jnp.float32),
        acc_sh=pltpu.VMEM_SHARED((ROWS_PER_CORE, D), jnp.float32),
        gsem=pltpu.SemaphoreType.DMA,
    ),
)
def _pool_sum(idx_hbm, sidx_hbm, table_hbm, out_hbm,
              idx_v, sidx_v, buf, zbuf, acc_sh, gsem):
    cid = lax.axis_index("c")
    sid = lax.axis_index("s")
    wid = cid * NS + sid

    pltpu.sync_copy(idx_hbm.at[wid], idx_v)
    pltpu.sync_copy(sidx_hbm.at[wid], sidx_v)

    def _zero(i, _):
        for dd in range(D // 16):
            zbuf[i, pl.ds(dd * 16, 16)] = jnp.zeros((16,), jnp.float32)
        return 0

    lax.fori_loop(0, RPW, _zero, 0)
    pltpu.sync_copy(zbuf, acc_sh.at[pl.ds(sid * RPW, RPW)])

    def _chunk(c, _):
        pltpu.async_copy(table_hbm.at[idx_v.at[c]], buf, gsem).wait()
        pltpu.sync_copy(buf, acc_sh.at[sidx_v.at[c]], add=True)
        return 0

    lax.fori_loop(0, NCHUNK, _chunk, 0)

    pltpu.sync_copy(acc_sh.at[pl.ds(sid * RPW, RPW)],
                    out_hbm.at[pl.ds(wid * RPW, RPW)])


def _mlp_body(p_ref, w1_ref, b1_ref, w2_ref, b2_ref, o_ref):
    h = jnp.dot(p_ref[...], w1_ref[...], preferred_element_type=jnp.float32)
    h = jnp.maximum(h + b1_ref[...], 0.0)
    o_ref[...] = (
        jnp.dot(h, w2_ref[...], preferred_element_type=jnp.float32)
        + b2_ref[...]
    )


def _mlp(pooled, w1, b1, w2, b2):
    return pl.pallas_call(
        _mlp_body,
        out_shape=jax.ShapeDtypeStruct((B, NCLASS), jnp.float32),
    )(pooled, w1, b1, w2, b2)


def kernel(x, table, W1, b1, W2, b2):
    idx = x.astype(jnp.int32).reshape(NW, NCHUNK, K)
    w_ = jnp.arange(NW, dtype=jnp.int32).reshape(NW, 1, 1) % NS
    c_ = jnp.arange(NCHUNK, dtype=jnp.int32).reshape(1, NCHUNK, 1)
    j_ = jnp.arange(K, dtype=jnp.int32).reshape(1, 1, K)
    sidx = w_ * RPW + c_ * RCHUNK + j_ // SEQ

    pooled_sum = _pool_sum(idx, sidx, table)
    return _mlp(pooled_sum, W1 * (1.0 / SEQ), b1.reshape(1, HIDDEN),
                W2, b2.reshape(1, NCLASS))
